# Initial kernel scaffold; baseline (speedup 1.0000x reference)
import functools

import jax
import jax.numpy as jnp
from jax import lax
from jax.experimental import pallas as pl
from jax.experimental.pallas import tpu as pltpu
from jax.experimental.pallas import tpu_sc as plsc

B = 4096
H = 50
HP = 52
G = 8
D = 64
NW = 32
NB = B // NW
HCB = 2
HCHUNKS = NB // HCB
GCB = 16
GCHUNKS = NB // GCB


def _sc_body(uidx_h, iidx_h, hidx_h, gidx_h, utab, htab, itab, gtab,
             uemb_o, hsum_o, iemb_o, gsum_o,
             uidx_v, iidx_v, hidx_v, gidx_v,
             urows_v, irows_v, hrows_v, grows_v, hsum_v, gsum_v, sem):
    wid = lax.axis_index("s") * 2 + lax.axis_index("c")

    pltpu.sync_copy(uidx_h.at[wid], uidx_v)
    pltpu.sync_copy(iidx_h.at[wid], iidx_v)
    pltpu.sync_copy(hidx_h.at[wid], hidx_v)
    pltpu.sync_copy(gidx_h.at[wid], gidx_v)

    pltpu.async_copy(utab.at[uidx_v], urows_v, sem).wait()
    pltpu.async_copy(itab.at[iidx_v], irows_v, sem).wait()
    pltpu.sync_copy(urows_v, uemb_o.at[wid])
    pltpu.sync_copy(irows_v, iemb_o.at[wid])

    zero4 = tuple(jnp.zeros((16,), jnp.float32) for _ in range(4))

    def hist_chunk(c, carry):
        pltpu.async_copy(htab.at[hidx_v.at[c]], hrows_v, sem).wait()
        for b in range(HCB):
            def hbody(h, accs):
                r = b * HP + h
                return tuple(accs[dg] + hrows_v[r, pl.ds(dg * 16, 16)]
                             for dg in range(4))
            accs = lax.fori_loop(0, HP, hbody, zero4)
            row = c * HCB + b
            for dg in range(4):
                hsum_v[row, pl.ds(dg * 16, 16)] = accs[dg]
        return carry

    lax.fori_loop(0, HCHUNKS, hist_chunk, 0)

    def genre_chunk(c, carry):
        pltpu.async_copy(gtab.at[gidx_v.at[c]], grows_v, sem).wait()

        def gouter(b, carry2):
            def gbody(g, accs):
                r = b * G + g
                return tuple(accs[dg] + grows_v[r, pl.ds(dg * 16, 16)]
                             for dg in range(4))
            accs = lax.fori_loop(0, G, gbody, zero4)
            row = c * GCB + b
            for dg in range(4):
                gsum_v[row, pl.ds(dg * 16, 16)] = accs[dg]
            return carry2

        lax.fori_loop(0, GCB, gouter, 0)
        return carry

    lax.fori_loop(0, GCHUNKS, genre_chunk, 0)

    pltpu.sync_copy(hsum_v, hsum_o.at[wid])
    pltpu.sync_copy(gsum_v, gsum_o.at[wid])


@jax.jit
def _sc_gather(uidx, iidx, hidx, gidx, utab, htab, itab, gtab):
    mesh = plsc.VectorSubcoreMesh(core_axis_name="c", subcore_axis_name="s")
    emb = jax.ShapeDtypeStruct((NW, NB, D), jnp.float32)
    f = functools.partial(
        pl.kernel,
        out_type=[emb, emb, emb, emb],
        mesh=mesh,
        scratch_types=[
            pltpu.VMEM((NB,), jnp.int32),
            pltpu.VMEM((NB,), jnp.int32),
            pltpu.VMEM((HCHUNKS, HCB * HP), jnp.int32),
            pltpu.VMEM((GCHUNKS, GCB * G), jnp.int32),
            pltpu.VMEM((NB, D), jnp.float32),
            pltpu.VMEM((NB, D), jnp.float32),
            pltpu.VMEM((HCB * HP, D), jnp.float32),
            pltpu.VMEM((GCB * G, D), jnp.float32),
            pltpu.VMEM((NB, D), jnp.float32),
            pltpu.VMEM((NB, D), jnp.float32),
            pltpu.SemaphoreType.DMA,
        ],
    )(_sc_body)
    return f(uidx, iidx, hidx, gidx, utab, htab, itab, gtab)


def _tc_body(hidx, gidx, u, hs, it, gs,
             uw1, ub1, uw2, ub2, iw1, ib1, iw2, ib2, out):
    f32 = jnp.float32
    hcnt = jnp.sum((hidx[...] != 0).astype(f32), axis=1, keepdims=True) + 1e-8
    h = hs[...] / hcnt
    gcnt = jnp.sum((gidx[...] != 0).astype(f32), axis=1, keepdims=True) + 1e-8
    g = gs[...] / gcnt
    uc = jnp.concatenate([u[...], h], axis=1)
    uh = jnp.maximum(jnp.dot(uc, uw1[...], preferred_element_type=f32)
                     + ub1[...], 0.0)
    uv = jnp.dot(uh, uw2[...], preferred_element_type=f32) + ub2[...]
    ic = jnp.concatenate([it[...], g], axis=1)
    ih = jnp.maximum(jnp.dot(ic, iw1[...], preferred_element_type=f32)
                     + ib1[...], 0.0)
    iv = jnp.dot(ih, iw2[...], preferred_element_type=f32) + ib2[...]
    logits = jnp.sum(uv * iv, axis=1)
    out[0, :] = jax.nn.sigmoid(logits)


def _tc_dense(hi, gi, u, hs, it, gs, uw1, ub1, uw2, ub2, iw1, ib1, iw2, ib2):
    BT = 512
    n = B // BT
    row = lambda i: (i, 0)
    rep = lambda i: (0, 0)
    return pl.pallas_call(
        _tc_body,
        grid=(n,),
        in_specs=[
            pl.BlockSpec((BT, H), row),
            pl.BlockSpec((BT, G), row),
            pl.BlockSpec((BT, D), row),
            pl.BlockSpec((BT, D), row),
            pl.BlockSpec((BT, D), row),
            pl.BlockSpec((BT, D), row),
            pl.BlockSpec((2 * D, 128), rep),
            pl.BlockSpec((1, 128), rep),
            pl.BlockSpec((128, D), rep),
            pl.BlockSpec((1, D), rep),
            pl.BlockSpec((2 * D, 128), rep),
            pl.BlockSpec((1, 128), rep),
            pl.BlockSpec((128, D), rep),
            pl.BlockSpec((1, D), rep),
        ],
        out_specs=pl.BlockSpec((1, BT), row),
        out_shape=jax.ShapeDtypeStruct((n, BT), jnp.float32),
    )(hi, gi, u, hs, it, gs, uw1, ub1, uw2, ub2, iw1, ib1, iw2, ib2)


def kernel(user_indices, history_indices, item_indices, genre_indices,
           user_table, item_hist_table, item_table, genre_table,
           u_w1, u_b1, u_w2, u_b2, i_w1, i_b1, i_w2, i_b2):
    ui = user_indices.astype(jnp.int32).reshape(NW, NB)
    ii = item_indices.astype(jnp.int32).reshape(NW, NB)
    hi = history_indices.astype(jnp.int32)
    gi = genre_indices.astype(jnp.int32)
    hip = jnp.pad(hi, ((0, 0), (0, HP - H)))
    hi3 = hip.reshape(NW, HCHUNKS, HCB * HP)
    gi3 = gi.reshape(NW, GCHUNKS, GCB * G)
    uemb, hsum, iemb, gsum = _sc_gather(
        ui, ii, hi3, gi3, user_table, item_hist_table, item_table,
        genre_table)
    out = _tc_dense(
        hi, gi,
        uemb.reshape(B, D), hsum.reshape(B, D),
        iemb.reshape(B, D), gsum.reshape(B, D),
        u_w1, u_b1.reshape(1, 128), u_w2, u_b2.reshape(1, D),
        i_w1, i_b1.reshape(1, 128), i_w2, i_b2.reshape(1, D))
    return out.reshape(B)


# speedup vs baseline: 1.5358x; 1.5358x over previous
import functools

import jax
import jax.numpy as jnp
from jax import lax
from jax.experimental import pallas as pl
from jax.experimental.pallas import tpu as pltpu
from jax.experimental.pallas import tpu_sc as plsc

B = 4096
H = 50
HP = 52
G = 8
D = 64
NW = 32
NB = B // NW
HCB = 2
HCHUNKS = NB // HCB
GCB = 16
GCHUNKS = NB // GCB


def _sc_body(uidx_h, iidx_h, hidx_h, gidx_h, utab, htab, itab, gtab,
             uemb_o, hsum_o, iemb_o, gsum_o,
             uidx_v, iidx_v, hidx_v, gidx_v,
             urows_v, irows_v, hrows_v, grows_v, hsum_v, gsum_v, sem):
    wid = lax.axis_index("s") * 2 + lax.axis_index("c")

    pltpu.sync_copy(uidx_h.at[wid], uidx_v)
    pltpu.sync_copy(iidx_h.at[wid], iidx_v)
    pltpu.sync_copy(hidx_h.at[wid], hidx_v)
    pltpu.sync_copy(gidx_h.at[wid], gidx_v)

    pltpu.async_copy(utab.at[uidx_v], urows_v, sem).wait()
    pltpu.async_copy(itab.at[iidx_v], irows_v, sem).wait()
    pltpu.sync_copy(urows_v, uemb_o.at[wid])
    pltpu.sync_copy(irows_v, iemb_o.at[wid])

    zero4 = tuple(jnp.zeros((16,), jnp.float32) for _ in range(4))

    def hist_chunk(c, carry):
        pltpu.async_copy(htab.at[hidx_v.at[c]], hrows_v, sem).wait()
        for b in range(HCB):
            def hbody(h, accs):
                r = b * HP + h
                return tuple(accs[dg] + hrows_v[r, pl.ds(dg * 16, 16)]
                             for dg in range(4))
            accs = lax.fori_loop(0, HP, hbody, zero4)
            row = c * HCB + b
            for dg in range(4):
                hsum_v[row, pl.ds(dg * 16, 16)] = accs[dg]
        return carry

    lax.fori_loop(0, HCHUNKS, hist_chunk, 0)

    def genre_chunk(c, carry):
        pltpu.async_copy(gtab.at[gidx_v.at[c]], grows_v, sem).wait()

        def gouter(b, carry2):
            def gbody(g, accs):
                r = b * G + g
                return tuple(accs[dg] + grows_v[r, pl.ds(dg * 16, 16)]
                             for dg in range(4))
            accs = lax.fori_loop(0, G, gbody, zero4)
            row = c * GCB + b
            for dg in range(4):
                gsum_v[row, pl.ds(dg * 16, 16)] = accs[dg]
            return carry2

        lax.fori_loop(0, GCB, gouter, 0)
        return carry

    lax.fori_loop(0, GCHUNKS, genre_chunk, 0)

    pltpu.sync_copy(hsum_v, hsum_o.at[wid])
    pltpu.sync_copy(gsum_v, gsum_o.at[wid])


@jax.jit
def _sc_gather(uidx, iidx, hidx, gidx, utab, htab, itab, gtab):
    mesh = plsc.VectorSubcoreMesh(core_axis_name="c", subcore_axis_name="s")
    emb = jax.ShapeDtypeStruct((NW, NB, D), jnp.float32)
    f = functools.partial(
        pl.kernel,
        out_type=[emb, emb, emb, emb],
        mesh=mesh,
        compiler_params=pltpu.CompilerParams(use_tc_tiling_on_sc=False),
        scratch_types=[
            pltpu.VMEM((NB,), jnp.int32),
            pltpu.VMEM((NB,), jnp.int32),
            pltpu.VMEM((HCHUNKS, HCB * HP), jnp.int32),
            pltpu.VMEM((GCHUNKS, GCB * G), jnp.int32),
            pltpu.VMEM((NB, D), jnp.float32),
            pltpu.VMEM((NB, D), jnp.float32),
            pltpu.VMEM((HCB * HP, D), jnp.float32),
            pltpu.VMEM((GCB * G, D), jnp.float32),
            pltpu.VMEM((NB, D), jnp.float32),
            pltpu.VMEM((NB, D), jnp.float32),
            pltpu.SemaphoreType.DMA,
        ],
    )(_sc_body)
    return f(uidx, iidx, hidx, gidx, utab, htab, itab, gtab)


def _tc_body(hidx, gidx, u, hs, it, gs,
             uw1, ub1, uw2, ub2, iw1, ib1, iw2, ib2, out):
    f32 = jnp.float32
    hcnt = jnp.sum((hidx[...] != 0).astype(f32), axis=1, keepdims=True) + 1e-8
    h = hs[...] / hcnt
    gcnt = jnp.sum((gidx[...] != 0).astype(f32), axis=1, keepdims=True) + 1e-8
    g = gs[...] / gcnt
    uc = jnp.concatenate([u[...], h], axis=1)
    uh = jnp.maximum(jnp.dot(uc, uw1[...], preferred_element_type=f32)
                     + ub1[...], 0.0)
    uv = jnp.dot(uh, uw2[...], preferred_element_type=f32) + ub2[...]
    ic = jnp.concatenate([it[...], g], axis=1)
    ih = jnp.maximum(jnp.dot(ic, iw1[...], preferred_element_type=f32)
                     + ib1[...], 0.0)
    iv = jnp.dot(ih, iw2[...], preferred_element_type=f32) + ib2[...]
    logits = jnp.sum(uv * iv, axis=1)
    out[...] = jax.nn.sigmoid(logits)


def _tc_dense(hi, gi, u, hs, it, gs, uw1, ub1, uw2, ub2, iw1, ib1, iw2, ib2):
    BT = 512
    n = B // BT
    row = lambda i: (i, 0)
    rep = lambda i: (0, 0)
    return pl.pallas_call(
        _tc_body,
        grid=(n,),
        in_specs=[
            pl.BlockSpec((BT, H), row),
            pl.BlockSpec((BT, G), row),
            pl.BlockSpec((BT, D), row),
            pl.BlockSpec((BT, D), row),
            pl.BlockSpec((BT, D), row),
            pl.BlockSpec((BT, D), row),
            pl.BlockSpec((2 * D, 128), rep),
            pl.BlockSpec((1, 128), rep),
            pl.BlockSpec((128, D), rep),
            pl.BlockSpec((1, D), rep),
            pl.BlockSpec((2 * D, 128), rep),
            pl.BlockSpec((1, 128), rep),
            pl.BlockSpec((128, D), rep),
            pl.BlockSpec((1, D), rep),
        ],
        out_specs=pl.BlockSpec((BT,), lambda i: (i,)),
        out_shape=jax.ShapeDtypeStruct((B,), jnp.float32),
    )(hi, gi, u, hs, it, gs, uw1, ub1, uw2, ub2, iw1, ib1, iw2, ib2)


def kernel(user_indices, history_indices, item_indices, genre_indices,
           user_table, item_hist_table, item_table, genre_table,
           u_w1, u_b1, u_w2, u_b2, i_w1, i_b1, i_w2, i_b2):
    ui = user_indices.astype(jnp.int32).reshape(NW, NB)
    ii = item_indices.astype(jnp.int32).reshape(NW, NB)
    hi = history_indices.astype(jnp.int32)
    gi = genre_indices.astype(jnp.int32)
    hip = jnp.pad(hi, ((0, 0), (0, HP - H)))
    hi3 = hip.reshape(NW, HCHUNKS, HCB * HP)
    gi3 = gi.reshape(NW, GCHUNKS, GCB * G)
    uemb, hsum, iemb, gsum = _sc_gather(
        ui, ii, hi3, gi3, user_table, item_hist_table, item_table,
        genre_table)
    out = _tc_dense(
        hi, gi,
        uemb.reshape(B, D), hsum.reshape(B, D),
        iemb.reshape(B, D), gsum.reshape(B, D),
        u_w1, u_b1.reshape(1, 128), u_w2, u_b2.reshape(1, D),
        i_w1, i_b1.reshape(1, 128), i_w2, i_b2.reshape(1, D))
    return out


# speedup vs baseline: 2.1794x; 1.4191x over previous
import functools

import jax
import jax.numpy as jnp
from jax import lax
from jax.experimental import pallas as pl
from jax.experimental.pallas import tpu as pltpu
from jax.experimental.pallas import tpu_sc as plsc

B = 4096
H = 50
HP = 52
G = 8
D = 64
NW = 32
NB = B // NW
HCB = 2
HCHUNKS = NB // HCB
GCB = 16
GCHUNKS = NB // GCB


def _mesh():
    return plsc.VectorSubcoreMesh(core_axis_name="c", subcore_axis_name="s")


def _wid():
    return lax.axis_index("s") * 2 + lax.axis_index("c")


def _rows_body(uidx_h, iidx_h, utab, itab, uemb_o, iemb_o,
               uidx_s, iidx_s, urows_v, irows_v,
               semu, semi):
    wid = _wid()
    pltpu.sync_copy(uidx_h.at[wid], uidx_s)
    pltpu.sync_copy(iidx_h.at[wid], iidx_s)

    def fire(i, carry):
        uv = uidx_s[pl.ds(i * 16, 16)]
        iv = iidx_s[pl.ds(i * 16, 16)]
        for lane in range(16):
            s = uv[lane]
            pltpu.async_copy(
                utab.at[pl.ds(s, 1)], urows_v.at[pl.ds(i * 16 + lane, 1)],
                semu)
            t = iv[lane]
            pltpu.async_copy(
                itab.at[pl.ds(t, 1)], irows_v.at[pl.ds(i * 16 + lane, 1)],
                semi)
        return carry

    lax.fori_loop(0, NB // 16, fire, 0)

    def drain(i, carry):
        pltpu.make_async_copy(
            utab.at[pl.ds(0, 1)], urows_v.at[pl.ds(i, 1)], semu).wait()
        pltpu.make_async_copy(
            itab.at[pl.ds(0, 1)], irows_v.at[pl.ds(i, 1)], semi).wait()
        return carry

    lax.fori_loop(0, NB, drain, 0)
    pltpu.sync_copy(urows_v, uemb_o.at[wid])
    pltpu.sync_copy(irows_v, iemb_o.at[wid])


@jax.jit
def _sc_rows(uidx, iidx, utab, itab):
    emb = jax.ShapeDtypeStruct((NW, NB, D), jnp.float32)
    f = functools.partial(
        pl.kernel,
        out_type=[emb, emb],
        mesh=_mesh(),
        scratch_types=[
            pltpu.VMEM((NB,), jnp.int32),
            pltpu.VMEM((NB,), jnp.int32),
            pltpu.VMEM((NB, D), jnp.float32),
            pltpu.VMEM((NB, D), jnp.float32),
            pltpu.SemaphoreType.DMA,
            pltpu.SemaphoreType.DMA,
        ],
    )(_rows_body)
    return f(uidx, iidx, utab, itab)


def _pool_body(hidx_h, gidx_h, htab, gtab, hsum_o, gsum_o,
               hidx_v, gidx_v, hrows0, hrows1, grows0, grows1,
               hsum_v, gsum_v, semh0, semh1, semg0, semg1):
    wid = _wid()
    pltpu.sync_copy(hidx_h.at[wid], hidx_v)
    pltpu.sync_copy(gidx_h.at[wid], gidx_v)

    hrows = (hrows0, hrows1)
    hsems = (semh0, semh1)
    grows = (grows0, grows1)
    gsems = (semg0, semg1)

    zero4 = tuple(jnp.zeros((16,), jnp.float32) for _ in range(4))

    pltpu.async_copy(htab.at[hidx_v.at[0]], hrows0, semh0)
    pltpu.async_copy(gtab.at[gidx_v.at[0]], grows0, semg0)

    def hist_group(c2, carry):
        for sub in range(2):
            c = c2 * 2 + sub
            pltpu.make_async_copy(
                htab.at[pl.ds(0, HCB * HP)], hrows[sub], hsems[sub]).wait()

            @pl.when(c + 1 < HCHUNKS)
            def _():
                pltpu.async_copy(
                    htab.at[hidx_v.at[c + 1]], hrows[1 - sub],
                    hsems[1 - sub])

            buf = hrows[sub]
            for b in range(HCB):
                def hbody(h4, accs):
                    a = list(accs)
                    for hh in range(4):
                        r = b * HP + h4 * 4 + hh
                        for dg in range(4):
                            a[dg] = a[dg] + buf[r, pl.ds(dg * 16, 16)]
                    return tuple(a)
                accs = lax.fori_loop(0, HP // 4, hbody, zero4)
                row = c * HCB + b
                for dg in range(4):
                    hsum_v[row, pl.ds(dg * 16, 16)] = accs[dg]
        return carry

    lax.fori_loop(0, HCHUNKS // 2, hist_group, 0)

    def genre_group(c2, carry):
        for sub in range(2):
            c = c2 * 2 + sub
            pltpu.make_async_copy(
                htab.at[pl.ds(0, GCB * G)], grows[sub], gsems[sub]).wait()

            @pl.when(c + 1 < GCHUNKS)
            def _():
                pltpu.async_copy(
                    gtab.at[gidx_v.at[c + 1]], grows[1 - sub],
                    gsems[1 - sub])

            buf = grows[sub]

            def gouter(b, carry2):
                def gbody(g, accs):
                    a = list(accs)
                    for gg in range(4):
                        r = b * G + g * 4 + gg
                        for dg in range(4):
                            a[dg] = a[dg] + buf[r, pl.ds(dg * 16, 16)]
                    return tuple(a)
                accs = lax.fori_loop(0, G // 4, gbody, zero4)
                row = c * GCB + b
                for dg in range(4):
                    gsum_v[row, pl.ds(dg * 16, 16)] = accs[dg]
                return carry2

            lax.fori_loop(0, GCB, gouter, 0)
        return carry

    lax.fori_loop(0, GCHUNKS // 2, genre_group, 0)

    pltpu.sync_copy(hsum_v, hsum_o.at[wid])
    pltpu.sync_copy(gsum_v, gsum_o.at[wid])


@jax.jit
def _sc_pool(hidx, gidx, htab, gtab):
    emb = jax.ShapeDtypeStruct((NW, NB, D), jnp.float32)
    f = functools.partial(
        pl.kernel,
        out_type=[emb, emb],
        mesh=_mesh(),
        compiler_params=pltpu.CompilerParams(use_tc_tiling_on_sc=False),
        scratch_types=[
            pltpu.VMEM((HCHUNKS, HCB * HP), jnp.int32),
            pltpu.VMEM((GCHUNKS, GCB * G), jnp.int32),
            pltpu.VMEM((HCB * HP, D), jnp.float32),
            pltpu.VMEM((HCB * HP, D), jnp.float32),
            pltpu.VMEM((GCB * G, D), jnp.float32),
            pltpu.VMEM((GCB * G, D), jnp.float32),
            pltpu.VMEM((NB, D), jnp.float32),
            pltpu.VMEM((NB, D), jnp.float32),
            pltpu.SemaphoreType.DMA,
            pltpu.SemaphoreType.DMA,
            pltpu.SemaphoreType.DMA,
            pltpu.SemaphoreType.DMA,
        ],
    )(_pool_body)
    return f(hidx, gidx, htab, gtab)


def _tc_body(hidx, gidx, u, hs, it, gs,
             uw1, ub1, uw2, ub2, iw1, ib1, iw2, ib2, out):
    f32 = jnp.float32
    hcnt = jnp.sum((hidx[...] != 0).astype(f32), axis=1, keepdims=True) + 1e-8
    h = hs[...] / hcnt
    gcnt = jnp.sum((gidx[...] != 0).astype(f32), axis=1, keepdims=True) + 1e-8
    g = gs[...] / gcnt
    uc = jnp.concatenate([u[...], h], axis=1)
    uh = jnp.maximum(jnp.dot(uc, uw1[...], preferred_element_type=f32)
                     + ub1[...], 0.0)
    uv = jnp.dot(uh, uw2[...], preferred_element_type=f32) + ub2[...]
    ic = jnp.concatenate([it[...], g], axis=1)
    ih = jnp.maximum(jnp.dot(ic, iw1[...], preferred_element_type=f32)
                     + ib1[...], 0.0)
    iv = jnp.dot(ih, iw2[...], preferred_element_type=f32) + ib2[...]
    logits = jnp.sum(uv * iv, axis=1)
    out[...] = jax.nn.sigmoid(logits)


def _tc_dense(hi, gi, u, hs, it, gs, uw1, ub1, uw2, ub2, iw1, ib1, iw2, ib2):
    BT = 512
    n = B // BT
    row = lambda i: (i, 0)
    rep = lambda i: (0, 0)
    return pl.pallas_call(
        _tc_body,
        grid=(n,),
        in_specs=[
            pl.BlockSpec((BT, H), row),
            pl.BlockSpec((BT, G), row),
            pl.BlockSpec((BT, D), row),
            pl.BlockSpec((BT, D), row),
            pl.BlockSpec((BT, D), row),
            pl.BlockSpec((BT, D), row),
            pl.BlockSpec((2 * D, 128), rep),
            pl.BlockSpec((1, 128), rep),
            pl.BlockSpec((128, D), rep),
            pl.BlockSpec((1, D), rep),
            pl.BlockSpec((2 * D, 128), rep),
            pl.BlockSpec((1, 128), rep),
            pl.BlockSpec((128, D), rep),
            pl.BlockSpec((1, D), rep),
        ],
        out_specs=pl.BlockSpec((BT,), lambda i: (i,)),
        out_shape=jax.ShapeDtypeStruct((B,), jnp.float32),
    )(hi, gi, u, hs, it, gs, uw1, ub1, uw2, ub2, iw1, ib1, iw2, ib2)


def kernel(user_indices, history_indices, item_indices, genre_indices,
           user_table, item_hist_table, item_table, genre_table,
           u_w1, u_b1, u_w2, u_b2, i_w1, i_b1, i_w2, i_b2):
    ui = user_indices.astype(jnp.int32).reshape(NW, NB)
    ii = item_indices.astype(jnp.int32).reshape(NW, NB)
    hi = history_indices.astype(jnp.int32)
    gi = genre_indices.astype(jnp.int32)
    hip = jnp.pad(hi, ((0, 0), (0, HP - H)))
    hi3 = hip.reshape(NW, HCHUNKS, HCB * HP)
    gi3 = gi.reshape(NW, GCHUNKS, GCB * G)
    uemb, iemb = _sc_rows(ui, ii, user_table, item_table)
    hsum, gsum = _sc_pool(hi3, gi3, item_hist_table, genre_table)
    out = _tc_dense(
        hi, gi,
        uemb.reshape(B, D), hsum.reshape(B, D),
        iemb.reshape(B, D), gsum.reshape(B, D),
        u_w1, u_b1.reshape(1, 128), u_w2, u_b2.reshape(1, D),
        i_w1, i_b1.reshape(1, 128), i_w2, i_b2.reshape(1, D))
    return out


# speedup vs baseline: 2.3806x; 1.0923x over previous
import functools

import jax
import jax.numpy as jnp
from jax import lax
from jax.experimental import pallas as pl
from jax.experimental.pallas import tpu as pltpu
from jax.experimental.pallas import tpu_sc as plsc

B = 4096
H = 50
HP = 52
G = 8
D = 64
NW = 32
NB = B // NW
HCB = 2
HCHUNKS = NB // HCB
GENRE_ROWS = 21


def _mesh():
    return plsc.VectorSubcoreMesh(core_axis_name="c", subcore_axis_name="s")


def _wid():
    return lax.axis_index("s") * 2 + lax.axis_index("c")


def _rows_body(uidx_h, iidx_h, utab, itab, uemb_o, iemb_o,
               uidx_s, iidx_s, urows_v, irows_v,
               semu, semi):
    wid = _wid()
    pltpu.sync_copy(uidx_h.at[wid], uidx_s)
    pltpu.sync_copy(iidx_h.at[wid], iidx_s)

    def fire(i, carry):
        uv = uidx_s[pl.ds(i * 16, 16)]
        iv = iidx_s[pl.ds(i * 16, 16)]
        for lane in range(16):
            s = uv[lane]
            pltpu.async_copy(
                utab.at[pl.ds(s, 1)], urows_v.at[pl.ds(i * 16 + lane, 1)],
                semu)
            t = iv[lane]
            pltpu.async_copy(
                itab.at[pl.ds(t, 1)], irows_v.at[pl.ds(i * 16 + lane, 1)],
                semi)
        return carry

    lax.fori_loop(0, NB // 16, fire, 0)

    def drain(i, carry):
        pltpu.make_async_copy(
            utab.at[pl.ds(0, 1)], urows_v.at[pl.ds(i, 1)], semu).wait()
        pltpu.make_async_copy(
            itab.at[pl.ds(0, 1)], irows_v.at[pl.ds(i, 1)], semi).wait()
        return carry

    lax.fori_loop(0, NB, drain, 0)
    pltpu.sync_copy(urows_v, uemb_o.at[wid])
    pltpu.sync_copy(irows_v, iemb_o.at[wid])


@jax.jit
def _sc_rows(uidx, iidx, utab, itab):
    emb = jax.ShapeDtypeStruct((NW, NB, D), jnp.float32)
    f = functools.partial(
        pl.kernel,
        out_type=[emb, emb],
        mesh=_mesh(),
        scratch_types=[
            pltpu.VMEM((NB,), jnp.int32),
            pltpu.VMEM((NB,), jnp.int32),
            pltpu.VMEM((NB, D), jnp.float32),
            pltpu.VMEM((NB, D), jnp.float32),
            pltpu.SemaphoreType.DMA,
            pltpu.SemaphoreType.DMA,
        ],
    )(_rows_body)
    return f(uidx, iidx, utab, itab)


NBUF = 8


def _pool_body(hidx_h, htab, hsum_o,
               hidx_v,
               h0, h1, h2, h3, h4, h5, h6, h7,
               hsum_v,
               s0, s1, s2, s3, s4, s5, s6, s7):
    wid = _wid()
    pltpu.sync_copy(hidx_h.at[wid], hidx_v)

    hrows = (h0, h1, h2, h3, h4, h5, h6, h7)
    hsems = (s0, s1, s2, s3, s4, s5, s6, s7)

    for k in range(NBUF):
        pltpu.async_copy(htab.at[hidx_v.at[k]], hrows[k], hsems[k])

    zero4 = tuple(jnp.zeros((16,), jnp.float32) for _ in range(4))

    def hist_group(g2, carry):
        for sub in range(NBUF):
            c = g2 * NBUF + sub
            pltpu.make_async_copy(
                htab.at[pl.ds(0, HCB * HP)], hrows[sub], hsems[sub]).wait()

            buf = hrows[sub]
            for b in range(HCB):
                def hbody(h4i, accs):
                    a = list(accs)
                    for hh in range(4):
                        r = b * HP + h4i * 4 + hh
                        for dg in range(4):
                            a[dg] = a[dg] + buf[r, pl.ds(dg * 16, 16)]
                    return tuple(a)
                accs = lax.fori_loop(0, HP // 4, hbody, zero4)
                row = c * HCB + b
                for dg in range(4):
                    hsum_v[row, pl.ds(dg * 16, 16)] = accs[dg]

            @pl.when(c + NBUF < HCHUNKS)
            def _():
                pltpu.async_copy(
                    htab.at[hidx_v.at[c + NBUF]], hrows[sub], hsems[sub])
        return carry

    lax.fori_loop(0, HCHUNKS // NBUF, hist_group, 0)

    pltpu.sync_copy(hsum_v, hsum_o.at[wid])


@jax.jit
def _sc_pool(hidx, htab):
    f = functools.partial(
        pl.kernel,
        out_type=[jax.ShapeDtypeStruct((NW, NB, D), jnp.float32)],
        mesh=_mesh(),
        compiler_params=pltpu.CompilerParams(use_tc_tiling_on_sc=False),
        scratch_types=(
            [pltpu.VMEM((HCHUNKS, HCB * HP), jnp.int32)]
            + [pltpu.VMEM((HCB * HP, D), jnp.float32)] * NBUF
            + [pltpu.VMEM((NB, D), jnp.float32)]
            + [pltpu.SemaphoreType.DMA] * NBUF
        ),
    )(_pool_body)
    return f(hidx, htab)[0]


def _tc_body(hidx, gidx, u, hs, it, gtab,
             uw1, ub1, uw2, ub2, iw1, ib1, iw2, ib2, out):
    f32 = jnp.float32
    hcnt = jnp.sum((hidx[...] != 0).astype(f32), axis=1, keepdims=True) + 1e-8
    h = hs[...] / hcnt
    gidxv = gidx[...]
    gcnt = jnp.sum((gidxv != 0).astype(f32), axis=1, keepdims=True) + 1e-8
    tio = lax.broadcasted_iota(jnp.int32, (1, GENRE_ROWS), 1)
    m = (gidxv[:, 0:1] == tio).astype(f32)
    for gg in range(1, G):
        m = m + (gidxv[:, gg:gg + 1] == tio).astype(f32)
    gsum = jnp.dot(m, gtab[...], preferred_element_type=f32)
    g = gsum / gcnt
    uc = jnp.concatenate([u[...], h], axis=1)
    uh = jnp.maximum(jnp.dot(uc, uw1[...], preferred_element_type=f32)
                     + ub1[...], 0.0)
    uv = jnp.dot(uh, uw2[...], preferred_element_type=f32) + ub2[...]
    ic = jnp.concatenate([it[...], g], axis=1)
    ih = jnp.maximum(jnp.dot(ic, iw1[...], preferred_element_type=f32)
                     + ib1[...], 0.0)
    iv = jnp.dot(ih, iw2[...], preferred_element_type=f32) + ib2[...]
    logits = jnp.sum(uv * iv, axis=1)
    out[...] = jax.nn.sigmoid(logits)


def _tc_dense(hi, gi, u, hs, it, gtab, uw1, ub1, uw2, ub2, iw1, ib1, iw2,
              ib2):
    BT = 512
    n = B // BT
    row = lambda i: (i, 0)
    rep = lambda i: (0, 0)
    return pl.pallas_call(
        _tc_body,
        grid=(n,),
        in_specs=[
            pl.BlockSpec((BT, H), row),
            pl.BlockSpec((BT, G), row),
            pl.BlockSpec((BT, D), row),
            pl.BlockSpec((BT, D), row),
            pl.BlockSpec((BT, D), row),
            pl.BlockSpec((GENRE_ROWS, D), rep),
            pl.BlockSpec((2 * D, 128), rep),
            pl.BlockSpec((1, 128), rep),
            pl.BlockSpec((128, D), rep),
            pl.BlockSpec((1, D), rep),
            pl.BlockSpec((2 * D, 128), rep),
            pl.BlockSpec((1, 128), rep),
            pl.BlockSpec((128, D), rep),
            pl.BlockSpec((1, D), rep),
        ],
        out_specs=pl.BlockSpec((BT,), lambda i: (i,)),
        out_shape=jax.ShapeDtypeStruct((B,), jnp.float32),
    )(hi, gi, u, hs, it, gtab, uw1, ub1, uw2, ub2, iw1, ib1, iw2, ib2)


def kernel(user_indices, history_indices, item_indices, genre_indices,
           user_table, item_hist_table, item_table, genre_table,
           u_w1, u_b1, u_w2, u_b2, i_w1, i_b1, i_w2, i_b2):
    ui = user_indices.astype(jnp.int32).reshape(NW, NB)
    ii = item_indices.astype(jnp.int32).reshape(NW, NB)
    hi = history_indices.astype(jnp.int32)
    gi = genre_indices.astype(jnp.int32)
    hip = jnp.pad(hi, ((0, 0), (0, HP - H)))
    hi3 = hip.reshape(NW, HCHUNKS, HCB * HP)
    uemb, iemb = _sc_rows(ui, ii, user_table, item_table)
    hsum = _sc_pool(hi3, item_hist_table)
    out = _tc_dense(
        hi, gi,
        uemb.reshape(B, D), hsum.reshape(B, D),
        iemb.reshape(B, D), genre_table,
        u_w1, u_b1.reshape(1, 128), u_w2, u_b2.reshape(1, D),
        i_w1, i_b1.reshape(1, 128), i_w2, i_b2.reshape(1, D))
    return out


# speedup vs baseline: 2.5320x; 1.0636x over previous
import functools

import jax
import jax.numpy as jnp
from jax import lax
from jax.experimental import pallas as pl
from jax.experimental.pallas import tpu as pltpu
from jax.experimental.pallas import tpu_sc as plsc

B = 4096
H = 50
HP = 52
G = 8
D = 64
NW = 32
NB = B // NW
HCB = 2
HCHUNKS = NB // HCB
GENRE_ROWS = 21


def _mesh():
    return plsc.VectorSubcoreMesh(core_axis_name="c", subcore_axis_name="s")


def _wid():
    return lax.axis_index("s") * 2 + lax.axis_index("c")


RPS = 8


def _tcg_body(uidx_s, iidx_s, *refs):
    ublks = refs[:RPS]
    iblks = refs[RPS:2 * RPS]
    uout, iout = refs[2 * RPS], refs[2 * RPS + 1]
    j = pl.program_id(0)
    lane = lax.broadcasted_iota(jnp.int32, (D, 128), 1)
    ucols = []
    icols = []
    for r in range(RPS):
        cu = uidx_s[j * RPS + r] % 128
        ucols.append(jnp.sum(jnp.where(lane == cu, ublks[r][...], 0.0),
                             axis=1, keepdims=True))
        ci = iidx_s[j * RPS + r] % 128
        icols.append(jnp.sum(jnp.where(lane == ci, iblks[r][...], 0.0),
                             axis=1, keepdims=True))
    uout[...] = jnp.transpose(jnp.concatenate(ucols, axis=1), (1, 0))
    iout[...] = jnp.transpose(jnp.concatenate(icols, axis=1), (1, 0))


def _tc_rows(uidx, iidx, utabT, itabT):
    nsteps = B // RPS

    def tab_spec(which, r):
        if which == 0:
            return pl.BlockSpec(
                (D, 128), lambda j, us, is_, r=r: (0, us[j * RPS + r] // 128))
        return pl.BlockSpec(
            (D, 128), lambda j, us, is_, r=r: (0, is_[j * RPS + r] // 128))

    grid_spec = pltpu.PrefetchScalarGridSpec(
        num_scalar_prefetch=2,
        grid=(nsteps,),
        in_specs=(
            [tab_spec(0, r) for r in range(RPS)]
            + [tab_spec(1, r) for r in range(RPS)]
        ),
        out_specs=[pl.BlockSpec((RPS, D), lambda j, us, is_: (j, 0)),
                   pl.BlockSpec((RPS, D), lambda j, us, is_: (j, 0))],
    )
    emb = jax.ShapeDtypeStruct((B, D), jnp.float32)
    return pl.pallas_call(
        _tcg_body,
        grid_spec=grid_spec,
        out_shape=[emb, emb],
    )(uidx, iidx, *([utabT] * RPS + [itabT] * RPS))


NBUF = 8


def _pool_body(hidx_h, htab, hsum_o,
               hidx_v,
               h0, h1, h2, h3, h4, h5, h6, h7,
               hsum_v,
               s0, s1, s2, s3, s4, s5, s6, s7):
    wid = _wid()
    pltpu.sync_copy(hidx_h.at[wid], hidx_v)

    hrows = (h0, h1, h2, h3, h4, h5, h6, h7)
    hsems = (s0, s1, s2, s3, s4, s5, s6, s7)

    for k in range(NBUF):
        pltpu.async_copy(htab.at[hidx_v.at[k]], hrows[k], hsems[k])

    zero4 = tuple(jnp.zeros((16,), jnp.float32) for _ in range(4))

    def hist_group(g2, carry):
        for sub in range(NBUF):
            c = g2 * NBUF + sub
            pltpu.make_async_copy(
                htab.at[pl.ds(0, HCB * HP)], hrows[sub], hsems[sub]).wait()

            buf = hrows[sub]
            for b in range(HCB):
                def hbody(h4i, accs):
                    a = list(accs)
                    for hh in range(4):
                        r = b * HP + h4i * 4 + hh
                        for dg in range(4):
                            a[dg] = a[dg] + buf[r, pl.ds(dg * 16, 16)]
                    return tuple(a)
                accs = lax.fori_loop(0, HP // 4, hbody, zero4)
                row = c * HCB + b
                for dg in range(4):
                    hsum_v[row, pl.ds(dg * 16, 16)] = accs[dg]

            @pl.when(c + NBUF < HCHUNKS)
            def _():
                pltpu.async_copy(
                    htab.at[hidx_v.at[c + NBUF]], hrows[sub], hsems[sub])
        return carry

    lax.fori_loop(0, HCHUNKS // NBUF, hist_group, 0)

    pltpu.sync_copy(hsum_v, hsum_o.at[wid])


@jax.jit
def _sc_pool(hidx, htab):
    f = functools.partial(
        pl.kernel,
        out_type=[jax.ShapeDtypeStruct((NW, NB, D), jnp.float32)],
        mesh=_mesh(),
        compiler_params=pltpu.CompilerParams(use_tc_tiling_on_sc=False),
        scratch_types=(
            [pltpu.VMEM((HCHUNKS, HCB * HP), jnp.int32)]
            + [pltpu.VMEM((HCB * HP, D), jnp.float32)] * NBUF
            + [pltpu.VMEM((NB, D), jnp.float32)]
            + [pltpu.SemaphoreType.DMA] * NBUF
        ),
    )(_pool_body)
    return f(hidx, htab)[0]


def _tc_body(hidx, gidx, u, hs, it, gtab,
             uw1, ub1, uw2, ub2, iw1, ib1, iw2, ib2, out):
    f32 = jnp.float32
    hcnt = jnp.sum((hidx[...] != 0).astype(f32), axis=1, keepdims=True) + 1e-8
    h = hs[...] / hcnt
    gidxv = gidx[...]
    gcnt = jnp.sum((gidxv != 0).astype(f32), axis=1, keepdims=True) + 1e-8
    tio = lax.broadcasted_iota(jnp.int32, (1, GENRE_ROWS), 1)
    m = (gidxv[:, 0:1] == tio).astype(f32)
    for gg in range(1, G):
        m = m + (gidxv[:, gg:gg + 1] == tio).astype(f32)
    gsum = jnp.dot(m, gtab[...], preferred_element_type=f32)
    g = gsum / gcnt
    uc = jnp.concatenate([u[...], h], axis=1)
    uh = jnp.maximum(jnp.dot(uc, uw1[...], preferred_element_type=f32)
                     + ub1[...], 0.0)
    uv = jnp.dot(uh, uw2[...], preferred_element_type=f32) + ub2[...]
    ic = jnp.concatenate([it[...], g], axis=1)
    ih = jnp.maximum(jnp.dot(ic, iw1[...], preferred_element_type=f32)
                     + ib1[...], 0.0)
    iv = jnp.dot(ih, iw2[...], preferred_element_type=f32) + ib2[...]
    logits = jnp.sum(uv * iv, axis=1)
    out[...] = jax.nn.sigmoid(logits)


def _tc_dense(hi, gi, u, hs, it, gtab, uw1, ub1, uw2, ub2, iw1, ib1, iw2,
              ib2):
    BT = 512
    n = B // BT
    row = lambda i: (i, 0)
    rep = lambda i: (0, 0)
    return pl.pallas_call(
        _tc_body,
        grid=(n,),
        in_specs=[
            pl.BlockSpec((BT, H), row),
            pl.BlockSpec((BT, G), row),
            pl.BlockSpec((BT, D), row),
            pl.BlockSpec((BT, D), row),
            pl.BlockSpec((BT, D), row),
            pl.BlockSpec((GENRE_ROWS, D), rep),
            pl.BlockSpec((2 * D, 128), rep),
            pl.BlockSpec((1, 128), rep),
            pl.BlockSpec((128, D), rep),
            pl.BlockSpec((1, D), rep),
            pl.BlockSpec((2 * D, 128), rep),
            pl.BlockSpec((1, 128), rep),
            pl.BlockSpec((128, D), rep),
            pl.BlockSpec((1, D), rep),
        ],
        out_specs=pl.BlockSpec((BT,), lambda i: (i,)),
        out_shape=jax.ShapeDtypeStruct((B,), jnp.float32),
    )(hi, gi, u, hs, it, gtab, uw1, ub1, uw2, ub2, iw1, ib1, iw2, ib2)


def kernel(user_indices, history_indices, item_indices, genre_indices,
           user_table, item_hist_table, item_table, genre_table,
           u_w1, u_b1, u_w2, u_b2, i_w1, i_b1, i_w2, i_b2):
    ui = user_indices.astype(jnp.int32)
    ii = item_indices.astype(jnp.int32)
    hi = history_indices.astype(jnp.int32)
    gi = genre_indices.astype(jnp.int32)
    hip = jnp.pad(hi, ((0, 0), (0, HP - H)))
    hi3 = hip.reshape(NW, HCHUNKS, HCB * HP)
    uemb, iemb = _tc_rows(ui, ii, user_table.T, item_table.T)
    hsum = _sc_pool(hi3, item_hist_table)
    out = _tc_dense(
        hi, gi,
        uemb, hsum.reshape(B, D),
        iemb, genre_table,
        u_w1, u_b1.reshape(1, 128), u_w2, u_b2.reshape(1, D),
        i_w1, i_b1.reshape(1, 128), i_w2, i_b2.reshape(1, D))
    return out


# speedup vs baseline: 2.6244x; 1.0365x over previous
import functools

import jax
import jax.numpy as jnp
from jax import lax
from jax.experimental import pallas as pl
from jax.experimental.pallas import tpu as pltpu
from jax.experimental.pallas import tpu_sc as plsc

B = 4096
H = 50
HP = 52
G = 8
D = 64
NW = 32
NB = B // NW
HCB = 2
HCHUNKS = NB // HCB
GENRE_ROWS = 21


def _mesh():
    return plsc.VectorSubcoreMesh(core_axis_name="c", subcore_axis_name="s")


def _wid():
    return lax.axis_index("s") * 2 + lax.axis_index("c")


RPS = 8


def _tcg_body(uidx_s, iidx_s, *refs):
    ublks = refs[:RPS]
    iblks = refs[RPS:2 * RPS]
    uout, iout = refs[2 * RPS], refs[2 * RPS + 1]
    j = pl.program_id(0)
    f32 = jnp.float32
    lane = lax.broadcasted_iota(jnp.int32, (RPS, RPS * 128), 1)
    subl = lax.broadcasted_iota(jnp.int32, (RPS, 1), 0)
    tu = jnp.zeros((RPS, 1), jnp.int32)
    ti = jnp.zeros((RPS, 1), jnp.int32)
    for r in range(RPS):
        tu = jnp.where(subl == r, uidx_s[j * RPS + r] % 128 + 128 * r, tu)
        ti = jnp.where(subl == r, iidx_s[j * RPS + r] % 128 + 128 * r, ti)
    ohu = (lane == tu).astype(f32)
    ohi = (lane == ti).astype(f32)
    cu = jnp.concatenate([ublks[r][...] for r in range(RPS)], axis=1)
    ci = jnp.concatenate([iblks[r][...] for r in range(RPS)], axis=1)
    dims = (((1,), (1,)), ((), ()))
    uout[...] = lax.dot_general(ohu, cu, dims, preferred_element_type=f32)
    iout[...] = lax.dot_general(ohi, ci, dims, preferred_element_type=f32)


def _tc_rows(uidx, iidx, utabT, itabT):
    nsteps = B // RPS

    def tab_spec(which, r):
        if which == 0:
            return pl.BlockSpec(
                (D, 128), lambda j, us, is_, r=r: (0, us[j * RPS + r] // 128))
        return pl.BlockSpec(
            (D, 128), lambda j, us, is_, r=r: (0, is_[j * RPS + r] // 128))

    grid_spec = pltpu.PrefetchScalarGridSpec(
        num_scalar_prefetch=2,
        grid=(nsteps,),
        in_specs=(
            [tab_spec(0, r) for r in range(RPS)]
            + [tab_spec(1, r) for r in range(RPS)]
        ),
        out_specs=[pl.BlockSpec((RPS, D), lambda j, us, is_: (j, 0)),
                   pl.BlockSpec((RPS, D), lambda j, us, is_: (j, 0))],
    )
    emb = jax.ShapeDtypeStruct((B, D), jnp.float32)
    return pl.pallas_call(
        _tcg_body,
        grid_spec=grid_spec,
        out_shape=[emb, emb],
    )(uidx, iidx, *([utabT] * RPS + [itabT] * RPS))


NBUF = 8


def _pool_body(hidx_h, htab, hsum_o,
               hidx_v,
               h0, h1, h2, h3, h4, h5, h6, h7,
               hsum_v,
               s0, s1, s2, s3, s4, s5, s6, s7):
    wid = _wid()
    pltpu.sync_copy(hidx_h.at[wid], hidx_v)

    hrows = (h0, h1, h2, h3, h4, h5, h6, h7)
    hsems = (s0, s1, s2, s3, s4, s5, s6, s7)

    for k in range(NBUF):
        pltpu.async_copy(htab.at[hidx_v.at[k]], hrows[k], hsems[k])

    zero4 = tuple(jnp.zeros((16,), jnp.float32) for _ in range(4))

    def hist_group(g2, carry):
        for sub in range(NBUF):
            c = g2 * NBUF + sub
            pltpu.make_async_copy(
                htab.at[pl.ds(0, HCB * HP)], hrows[sub], hsems[sub]).wait()

            buf = hrows[sub]
            for b in range(HCB):
                def hbody(h4i, accs):
                    a = list(accs)
                    for hh in range(4):
                        r = b * HP + h4i * 4 + hh
                        for dg in range(4):
                            a[dg] = a[dg] + buf[r, pl.ds(dg * 16, 16)]
                    return tuple(a)
                accs = lax.fori_loop(0, HP // 4, hbody, zero4)
                row = c * HCB + b
                for dg in range(4):
                    hsum_v[row, pl.ds(dg * 16, 16)] = accs[dg]

            @pl.when(c + NBUF < HCHUNKS)
            def _():
                pltpu.async_copy(
                    htab.at[hidx_v.at[c + NBUF]], hrows[sub], hsems[sub])
        return carry

    lax.fori_loop(0, HCHUNKS // NBUF, hist_group, 0)

    pltpu.sync_copy(hsum_v, hsum_o.at[wid])


@jax.jit
def _sc_pool(hidx, htab):
    f = functools.partial(
        pl.kernel,
        out_type=[jax.ShapeDtypeStruct((NW, NB, D), jnp.float32)],
        mesh=_mesh(),
        compiler_params=pltpu.CompilerParams(use_tc_tiling_on_sc=False),
        scratch_types=(
            [pltpu.VMEM((HCHUNKS, HCB * HP), jnp.int32)]
            + [pltpu.VMEM((HCB * HP, D), jnp.float32)] * NBUF
            + [pltpu.VMEM((NB, D), jnp.float32)]
            + [pltpu.SemaphoreType.DMA] * NBUF
        ),
    )(_pool_body)
    return f(hidx, htab)[0]


def _tc_body(hidx, gidx, u, hs, it, gtab,
             uw1, ub1, uw2, ub2, iw1, ib1, iw2, ib2, out):
    f32 = jnp.float32
    hcnt = jnp.sum((hidx[...] != 0).astype(f32), axis=1, keepdims=True) + 1e-8
    h = hs[...] / hcnt
    gidxv = gidx[...]
    gcnt = jnp.sum((gidxv != 0).astype(f32), axis=1, keepdims=True) + 1e-8
    tio = lax.broadcasted_iota(jnp.int32, (1, GENRE_ROWS), 1)
    m = (gidxv[:, 0:1] == tio).astype(f32)
    for gg in range(1, G):
        m = m + (gidxv[:, gg:gg + 1] == tio).astype(f32)
    gsum = jnp.dot(m, gtab[...], preferred_element_type=f32)
    g = gsum / gcnt
    uc = jnp.concatenate([u[...], h], axis=1)
    uh = jnp.maximum(jnp.dot(uc, uw1[...], preferred_element_type=f32)
                     + ub1[...], 0.0)
    uv = jnp.dot(uh, uw2[...], preferred_element_type=f32) + ub2[...]
    ic = jnp.concatenate([it[...], g], axis=1)
    ih = jnp.maximum(jnp.dot(ic, iw1[...], preferred_element_type=f32)
                     + ib1[...], 0.0)
    iv = jnp.dot(ih, iw2[...], preferred_element_type=f32) + ib2[...]
    logits = jnp.sum(uv * iv, axis=1)
    out[...] = jax.nn.sigmoid(logits)


def _tc_dense(hi, gi, u, hs, it, gtab, uw1, ub1, uw2, ub2, iw1, ib1, iw2,
              ib2):
    BT = 512
    n = B // BT
    row = lambda i: (i, 0)
    rep = lambda i: (0, 0)
    return pl.pallas_call(
        _tc_body,
        grid=(n,),
        in_specs=[
            pl.BlockSpec((BT, H), row),
            pl.BlockSpec((BT, G), row),
            pl.BlockSpec((BT, D), row),
            pl.BlockSpec((BT, D), row),
            pl.BlockSpec((BT, D), row),
            pl.BlockSpec((GENRE_ROWS, D), rep),
            pl.BlockSpec((2 * D, 128), rep),
            pl.BlockSpec((1, 128), rep),
            pl.BlockSpec((128, D), rep),
            pl.BlockSpec((1, D), rep),
            pl.BlockSpec((2 * D, 128), rep),
            pl.BlockSpec((1, 128), rep),
            pl.BlockSpec((128, D), rep),
            pl.BlockSpec((1, D), rep),
        ],
        out_specs=pl.BlockSpec((BT,), lambda i: (i,)),
        out_shape=jax.ShapeDtypeStruct((B,), jnp.float32),
    )(hi, gi, u, hs, it, gtab, uw1, ub1, uw2, ub2, iw1, ib1, iw2, ib2)


def kernel(user_indices, history_indices, item_indices, genre_indices,
           user_table, item_hist_table, item_table, genre_table,
           u_w1, u_b1, u_w2, u_b2, i_w1, i_b1, i_w2, i_b2):
    ui = user_indices.astype(jnp.int32)
    ii = item_indices.astype(jnp.int32)
    hi = history_indices.astype(jnp.int32)
    gi = genre_indices.astype(jnp.int32)
    hip = jnp.pad(hi, ((0, 0), (0, HP - H)))
    hi3 = hip.reshape(NW, HCHUNKS, HCB * HP)
    uemb, iemb = _tc_rows(ui, ii, user_table.T, item_table.T)
    hsum = _sc_pool(hi3, item_hist_table)
    out = _tc_dense(
        hi, gi,
        uemb, hsum.reshape(B, D),
        iemb, genre_table,
        u_w1, u_b1.reshape(1, 128), u_w2, u_b2.reshape(1, D),
        i_w1, i_b1.reshape(1, 128), i_w2, i_b2.reshape(1, D))
    return out


# speedup vs baseline: 3.0314x; 1.1551x over previous
import functools

import jax
import jax.numpy as jnp
from jax import lax
from jax.experimental import pallas as pl
from jax.experimental.pallas import tpu as pltpu
from jax.experimental.pallas import tpu_sc as plsc

B = 4096
H = 50
HP = 52
G = 8
D = 64
NW = 32
NB = B // NW
HCB = 2
HCHUNKS = NB // HCB
GENRE_ROWS = 21


def _mesh():
    return plsc.VectorSubcoreMesh(core_axis_name="c", subcore_axis_name="s")


def _wid():
    return lax.axis_index("s") * 2 + lax.axis_index("c")


RPS = 16


def _tcg_body(uidx_s, iidx_s, *refs):
    ublks = refs[:RPS]
    iblks = refs[RPS:2 * RPS]
    uout, iout = refs[2 * RPS], refs[2 * RPS + 1]
    j = pl.program_id(0)
    f32 = jnp.float32
    lane = lax.broadcasted_iota(jnp.int32, (RPS, RPS * 128), 1)
    subl = lax.broadcasted_iota(jnp.int32, (RPS, 1), 0)
    tu = jnp.zeros((RPS, 1), jnp.int32)
    ti = jnp.zeros((RPS, 1), jnp.int32)
    for r in range(RPS):
        tu = jnp.where(subl == r, uidx_s[j * RPS + r] % 128 + 128 * r, tu)
        ti = jnp.where(subl == r, iidx_s[j * RPS + r] % 128 + 128 * r, ti)
    ohu = (lane == tu).astype(f32)
    ohi = (lane == ti).astype(f32)
    cu = jnp.concatenate([ublks[r][...] for r in range(RPS)], axis=1)
    ci = jnp.concatenate([iblks[r][...] for r in range(RPS)], axis=1)
    dims = (((1,), (1,)), ((), ()))
    uout[...] = lax.dot_general(ohu, cu, dims, preferred_element_type=f32)
    iout[...] = lax.dot_general(ohi, ci, dims, preferred_element_type=f32)


def _tc_rows(uidx, iidx, utabT, itabT):
    nsteps = B // RPS

    def tab_spec(which, r):
        if which == 0:
            return pl.BlockSpec(
                (D, 128), lambda j, us, is_, r=r: (0, us[j * RPS + r] // 128))
        return pl.BlockSpec(
            (D, 128), lambda j, us, is_, r=r: (0, is_[j * RPS + r] // 128))

    grid_spec = pltpu.PrefetchScalarGridSpec(
        num_scalar_prefetch=2,
        grid=(nsteps,),
        in_specs=(
            [tab_spec(0, r) for r in range(RPS)]
            + [tab_spec(1, r) for r in range(RPS)]
        ),
        out_specs=[pl.BlockSpec((RPS, D), lambda j, us, is_: (j, 0)),
                   pl.BlockSpec((RPS, D), lambda j, us, is_: (j, 0))],
    )
    emb = jax.ShapeDtypeStruct((B, D), jnp.float32)
    return pl.pallas_call(
        _tcg_body,
        grid_spec=grid_spec,
        out_shape=[emb, emb],
    )(uidx, iidx, *([utabT] * RPS + [itabT] * RPS))


NBUF = 8


def _pool_body(hidx_h, htab, hsum_o,
               hidx_v,
               h0, h1, h2, h3, h4, h5, h6, h7,
               hsum_v,
               s0, s1, s2, s3, s4, s5, s6, s7):
    wid = _wid()
    pltpu.sync_copy(hidx_h.at[wid], hidx_v)

    hrows = (h0, h1, h2, h3, h4, h5, h6, h7)
    hsems = (s0, s1, s2, s3, s4, s5, s6, s7)

    for k in range(NBUF):
        pltpu.async_copy(htab.at[hidx_v.at[k]], hrows[k], hsems[k])

    zero4 = tuple(jnp.zeros((16,), jnp.float32) for _ in range(4))

    def hist_group(g2, carry):
        for sub in range(NBUF):
            c = g2 * NBUF + sub
            pltpu.make_async_copy(
                htab.at[pl.ds(0, HCB * HP)], hrows[sub], hsems[sub]).wait()

            buf = hrows[sub]
            for b in range(HCB):
                def hbody(h4i, accs):
                    a = list(accs)
                    for hh in range(4):
                        r = b * HP + h4i * 4 + hh
                        for dg in range(4):
                            a[dg] = a[dg] + buf[r, pl.ds(dg * 16, 16)]
                    return tuple(a)
                accs = lax.fori_loop(0, HP // 4, hbody, zero4)
                row = c * HCB + b
                for dg in range(4):
                    hsum_v[row, pl.ds(dg * 16, 16)] = accs[dg]

            @pl.when(c + NBUF < HCHUNKS)
            def _():
                pltpu.async_copy(
                    htab.at[hidx_v.at[c + NBUF]], hrows[sub], hsems[sub])
        return carry

    lax.fori_loop(0, HCHUNKS // NBUF, hist_group, 0)

    pltpu.sync_copy(hsum_v, hsum_o.at[wid])


@jax.jit
def _sc_pool(hidx, htab):
    f = functools.partial(
        pl.kernel,
        out_type=[jax.ShapeDtypeStruct((NW, NB, D), jnp.float32)],
        mesh=_mesh(),
        compiler_params=pltpu.CompilerParams(use_tc_tiling_on_sc=False),
        scratch_types=(
            [pltpu.VMEM((HCHUNKS, HCB * HP), jnp.int32)]
            + [pltpu.VMEM((HCB * HP, D), jnp.float32)] * NBUF
            + [pltpu.VMEM((NB, D), jnp.float32)]
            + [pltpu.SemaphoreType.DMA] * NBUF
        ),
    )(_pool_body)
    return f(hidx, htab)[0]


def _tc_body(hidx, gidx, u, hs, it, gtab,
             uw1, ub1, uw2, ub2, iw1, ib1, iw2, ib2, out):
    f32 = jnp.float32
    hcnt = jnp.sum((hidx[...] != 0).astype(f32), axis=1, keepdims=True) + 1e-8
    h = hs[...] / hcnt
    gidxv = gidx[...]
    gcnt = jnp.sum((gidxv != 0).astype(f32), axis=1, keepdims=True) + 1e-8
    tio = lax.broadcasted_iota(jnp.int32, (1, GENRE_ROWS), 1)
    m = (gidxv[:, 0:1] == tio).astype(f32)
    for gg in range(1, G):
        m = m + (gidxv[:, gg:gg + 1] == tio).astype(f32)
    gsum = jnp.dot(m, gtab[...], preferred_element_type=f32)
    g = gsum / gcnt
    uc = jnp.concatenate([u[...], h], axis=1)
    uh = jnp.maximum(jnp.dot(uc, uw1[...], preferred_element_type=f32)
                     + ub1[...], 0.0)
    uv = jnp.dot(uh, uw2[...], preferred_element_type=f32) + ub2[...]
    ic = jnp.concatenate([it[...], g], axis=1)
    ih = jnp.maximum(jnp.dot(ic, iw1[...], preferred_element_type=f32)
                     + ib1[...], 0.0)
    iv = jnp.dot(ih, iw2[...], preferred_element_type=f32) + ib2[...]
    logits = jnp.sum(uv * iv, axis=1)
    out[...] = jax.nn.sigmoid(logits)


def _tc_dense(hi, gi, u, hs, it, gtab, uw1, ub1, uw2, ub2, iw1, ib1, iw2,
              ib2):
    BT = 512
    n = B // BT
    row = lambda i: (i, 0)
    rep = lambda i: (0, 0)
    return pl.pallas_call(
        _tc_body,
        grid=(n,),
        in_specs=[
            pl.BlockSpec((BT, H), row),
            pl.BlockSpec((BT, G), row),
            pl.BlockSpec((BT, D), row),
            pl.BlockSpec((BT, D), row),
            pl.BlockSpec((BT, D), row),
            pl.BlockSpec((GENRE_ROWS, D), rep),
            pl.BlockSpec((2 * D, 128), rep),
            pl.BlockSpec((1, 128), rep),
            pl.BlockSpec((128, D), rep),
            pl.BlockSpec((1, D), rep),
            pl.BlockSpec((2 * D, 128), rep),
            pl.BlockSpec((1, 128), rep),
            pl.BlockSpec((128, D), rep),
            pl.BlockSpec((1, D), rep),
        ],
        out_specs=pl.BlockSpec((BT,), lambda i: (i,)),
        out_shape=jax.ShapeDtypeStruct((B,), jnp.float32),
    )(hi, gi, u, hs, it, gtab, uw1, ub1, uw2, ub2, iw1, ib1, iw2, ib2)


def kernel(user_indices, history_indices, item_indices, genre_indices,
           user_table, item_hist_table, item_table, genre_table,
           u_w1, u_b1, u_w2, u_b2, i_w1, i_b1, i_w2, i_b2):
    ui = user_indices.astype(jnp.int32)
    ii = item_indices.astype(jnp.int32)
    hi = history_indices.astype(jnp.int32)
    gi = genre_indices.astype(jnp.int32)
    hip = jnp.pad(hi, ((0, 0), (0, HP - H)))
    hi3 = hip.reshape(NW, HCHUNKS, HCB * HP)
    uemb, iemb = _tc_rows(ui, ii, user_table.T, item_table.T)
    hsum = _sc_pool(hi3, item_hist_table)
    out = _tc_dense(
        hi, gi,
        uemb, hsum.reshape(B, D),
        iemb, genre_table,
        u_w1, u_b1.reshape(1, 128), u_w2, u_b2.reshape(1, D),
        i_w1, i_b1.reshape(1, 128), i_w2, i_b2.reshape(1, D))
    return out


# speedup vs baseline: 3.1651x; 1.0441x over previous
import functools

import jax
import jax.numpy as jnp
from jax import lax
from jax.experimental import pallas as pl
from jax.experimental.pallas import tpu as pltpu
from jax.experimental.pallas import tpu_sc as plsc

B = 4096
H = 50
HP = 52
G = 8
D = 64
NW = 32
NB = B // NW
HCB = 2
HCHUNKS = NB // HCB
GENRE_ROWS = 21


def _mesh():
    return plsc.VectorSubcoreMesh(core_axis_name="c", subcore_axis_name="s")


def _wid():
    return lax.axis_index("s") * 2 + lax.axis_index("c")


RPS = 32


def _tcg_body(uidx_s, iidx_s, *refs):
    ublks = refs[:RPS]
    iblks = refs[RPS:2 * RPS]
    uout, iout = refs[2 * RPS], refs[2 * RPS + 1]
    j = pl.program_id(0)
    f32 = jnp.float32
    lane = lax.broadcasted_iota(jnp.int32, (RPS, RPS * 128), 1)
    subl = lax.broadcasted_iota(jnp.int32, (RPS, 1), 0)
    tu = jnp.zeros((RPS, 1), jnp.int32)
    ti = jnp.zeros((RPS, 1), jnp.int32)
    for r in range(RPS):
        tu = jnp.where(subl == r, uidx_s[j * RPS + r] % 128 + 128 * r, tu)
        ti = jnp.where(subl == r, iidx_s[j * RPS + r] % 128 + 128 * r, ti)
    ohu = (lane == tu).astype(f32)
    ohi = (lane == ti).astype(f32)
    cu = jnp.concatenate([ublks[r][...] for r in range(RPS)], axis=1)
    ci = jnp.concatenate([iblks[r][...] for r in range(RPS)], axis=1)
    dims = (((1,), (1,)), ((), ()))
    uout[...] = lax.dot_general(ohu, cu, dims, preferred_element_type=f32)
    iout[...] = lax.dot_general(ohi, ci, dims, preferred_element_type=f32)


def _tc_rows(uidx, iidx, utabT, itabT):
    nsteps = B // RPS

    def tab_spec(which, r):
        if which == 0:
            return pl.BlockSpec(
                (D, 128), lambda j, us, is_, r=r: (0, us[j * RPS + r] // 128))
        return pl.BlockSpec(
            (D, 128), lambda j, us, is_, r=r: (0, is_[j * RPS + r] // 128))

    grid_spec = pltpu.PrefetchScalarGridSpec(
        num_scalar_prefetch=2,
        grid=(nsteps,),
        in_specs=(
            [tab_spec(0, r) for r in range(RPS)]
            + [tab_spec(1, r) for r in range(RPS)]
        ),
        out_specs=[pl.BlockSpec((RPS, D), lambda j, us, is_: (j, 0)),
                   pl.BlockSpec((RPS, D), lambda j, us, is_: (j, 0))],
    )
    emb = jax.ShapeDtypeStruct((B, D), jnp.float32)
    return pl.pallas_call(
        _tcg_body,
        grid_spec=grid_spec,
        out_shape=[emb, emb],
    )(uidx, iidx, *([utabT] * RPS + [itabT] * RPS))


NBUF = 8


def _pool_body(hidx_h, htab, hsum_o,
               hidx_v,
               h0, h1, h2, h3, h4, h5, h6, h7,
               hsum_v,
               s0, s1, s2, s3, s4, s5, s6, s7):
    wid = _wid()
    pltpu.sync_copy(hidx_h.at[wid], hidx_v)

    hrows = (h0, h1, h2, h3, h4, h5, h6, h7)
    hsems = (s0, s1, s2, s3, s4, s5, s6, s7)

    for k in range(NBUF):
        pltpu.async_copy(htab.at[hidx_v.at[k]], hrows[k], hsems[k])

    zero4 = tuple(jnp.zeros((16,), jnp.float32) for _ in range(4))

    def hist_group(g2, carry):
        for sub in range(NBUF):
            c = g2 * NBUF + sub
            pltpu.make_async_copy(
                htab.at[pl.ds(0, HCB * HP)], hrows[sub], hsems[sub]).wait()

            buf = hrows[sub]
            for b in range(HCB):
                def hbody(h4i, accs):
                    a = list(accs)
                    for hh in range(4):
                        r = b * HP + h4i * 4 + hh
                        for dg in range(4):
                            a[dg] = a[dg] + buf[r, pl.ds(dg * 16, 16)]
                    return tuple(a)
                accs = lax.fori_loop(0, HP // 4, hbody, zero4)
                row = c * HCB + b
                for dg in range(4):
                    hsum_v[row, pl.ds(dg * 16, 16)] = accs[dg]

            @pl.when(c + NBUF < HCHUNKS)
            def _():
                pltpu.async_copy(
                    htab.at[hidx_v.at[c + NBUF]], hrows[sub], hsems[sub])
        return carry

    lax.fori_loop(0, HCHUNKS // NBUF, hist_group, 0)

    pltpu.sync_copy(hsum_v, hsum_o.at[wid])


@jax.jit
def _sc_pool(hidx, htab):
    f = functools.partial(
        pl.kernel,
        out_type=[jax.ShapeDtypeStruct((NW, NB, D), jnp.float32)],
        mesh=_mesh(),
        compiler_params=pltpu.CompilerParams(use_tc_tiling_on_sc=False),
        scratch_types=(
            [pltpu.VMEM((HCHUNKS, HCB * HP), jnp.int32)]
            + [pltpu.VMEM((HCB * HP, D), jnp.float32)] * NBUF
            + [pltpu.VMEM((NB, D), jnp.float32)]
            + [pltpu.SemaphoreType.DMA] * NBUF
        ),
    )(_pool_body)
    return f(hidx, htab)[0]


def _tc_body(hidx, gidx, u, hs, it, gtab,
             uw1, ub1, uw2, ub2, iw1, ib1, iw2, ib2, out):
    f32 = jnp.float32
    hcnt = jnp.sum((hidx[...] != 0).astype(f32), axis=1, keepdims=True) + 1e-8
    h = hs[...] / hcnt
    gidxv = gidx[...]
    gcnt = jnp.sum((gidxv != 0).astype(f32), axis=1, keepdims=True) + 1e-8
    tio = lax.broadcasted_iota(jnp.int32, (1, GENRE_ROWS), 1)
    m = (gidxv[:, 0:1] == tio).astype(f32)
    for gg in range(1, G):
        m = m + (gidxv[:, gg:gg + 1] == tio).astype(f32)
    gsum = jnp.dot(m, gtab[...], preferred_element_type=f32)
    g = gsum / gcnt
    uc = jnp.concatenate([u[...], h], axis=1)
    uh = jnp.maximum(jnp.dot(uc, uw1[...], preferred_element_type=f32)
                     + ub1[...], 0.0)
    uv = jnp.dot(uh, uw2[...], preferred_element_type=f32) + ub2[...]
    ic = jnp.concatenate([it[...], g], axis=1)
    ih = jnp.maximum(jnp.dot(ic, iw1[...], preferred_element_type=f32)
                     + ib1[...], 0.0)
    iv = jnp.dot(ih, iw2[...], preferred_element_type=f32) + ib2[...]
    logits = jnp.sum(uv * iv, axis=1)
    out[...] = jax.nn.sigmoid(logits)


def _tc_dense(hi, gi, u, hs, it, gtab, uw1, ub1, uw2, ub2, iw1, ib1, iw2,
              ib2):
    BT = 512
    n = B // BT
    row = lambda i: (i, 0)
    rep = lambda i: (0, 0)
    return pl.pallas_call(
        _tc_body,
        grid=(n,),
        in_specs=[
            pl.BlockSpec((BT, H), row),
            pl.BlockSpec((BT, G), row),
            pl.BlockSpec((BT, D), row),
            pl.BlockSpec((BT, D), row),
            pl.BlockSpec((BT, D), row),
            pl.BlockSpec((GENRE_ROWS, D), rep),
            pl.BlockSpec((2 * D, 128), rep),
            pl.BlockSpec((1, 128), rep),
            pl.BlockSpec((128, D), rep),
            pl.BlockSpec((1, D), rep),
            pl.BlockSpec((2 * D, 128), rep),
            pl.BlockSpec((1, 128), rep),
            pl.BlockSpec((128, D), rep),
            pl.BlockSpec((1, D), rep),
        ],
        out_specs=pl.BlockSpec((BT,), lambda i: (i,)),
        out_shape=jax.ShapeDtypeStruct((B,), jnp.float32),
    )(hi, gi, u, hs, it, gtab, uw1, ub1, uw2, ub2, iw1, ib1, iw2, ib2)


def kernel(user_indices, history_indices, item_indices, genre_indices,
           user_table, item_hist_table, item_table, genre_table,
           u_w1, u_b1, u_w2, u_b2, i_w1, i_b1, i_w2, i_b2):
    ui = user_indices.astype(jnp.int32)
    ii = item_indices.astype(jnp.int32)
    hi = history_indices.astype(jnp.int32)
    gi = genre_indices.astype(jnp.int32)
    hip = jnp.pad(hi, ((0, 0), (0, HP - H)))
    hi3 = hip.reshape(NW, HCHUNKS, HCB * HP)
    uemb, iemb = _tc_rows(ui, ii, user_table.T, item_table.T)
    hsum = _sc_pool(hi3, item_hist_table)
    out = _tc_dense(
        hi, gi,
        uemb, hsum.reshape(B, D),
        iemb, genre_table,
        u_w1, u_b1.reshape(1, 128), u_w2, u_b2.reshape(1, D),
        i_w1, i_b1.reshape(1, 128), i_w2, i_b2.reshape(1, D))
    return out
